# baseline (speedup 1.0000x reference)
import jax
import jax.numpy as jnp
from jax import lax
from jax.experimental import pallas as pl
from jax.experimental.pallas import tpu as pltpu
from jax.experimental.pallas import tpu_sc as plsc

N = 10000
E = 320000
D = 128
DH = D // 2
NC = 2
NS = 16
EP = 327680
EPS = EP // NS
K = 128
C = EPS // K
NP = 10112
RPT = NP // NS
DW = 8
_NBUF = 4
_PF = 2

_mesh = plsc.VectorSubcoreMesh(core_axis_name="c", subcore_axis_name="s",
                               num_cores=NC, num_subcores=NS)
_sc_params = pltpu.CompilerParams(use_tc_tiling_on_sc=False)


def _sc_layer(with_deg):
    out_type = [jax.ShapeDtypeStruct((NP, D), jnp.float32)]
    scratch = [
        pltpu.VMEM((C, K), jnp.int32),
        pltpu.VMEM((C, K), jnp.int32),
    ] + [pltpu.VMEM((K, DH), jnp.float32)] * _NBUF + [
        pltpu.VMEM_SHARED((NP, DH), jnp.float32),
    ] + [pltpu.SemaphoreType.DMA] * (2 * _NBUF)
    if with_deg:
        out_type.append(jax.ShapeDtypeStruct((NP, DW), jnp.float32))
        scratch += [
            pltpu.VMEM((K, DW), jnp.float32),
            pltpu.VMEM_SHARED((NP, DW), jnp.float32),
        ] + [pltpu.SemaphoreType.DMA] * _NBUF

    def body(y, srcL_r, srcR_r, dst_r, z, *rest):
        nb_ = _NBUF
        if with_deg:
            (z8, ones, agg_out, deg_out, idx_s, idx_d) = rest[:6]
            bufs = list(rest[6:6 + nb_])
            acc = rest[6 + nb_]
            gsem = list(rest[7 + nb_:7 + 2 * nb_])
            ssem = list(rest[7 + 2 * nb_:7 + 3 * nb_])
            ones_v = rest[7 + 3 * nb_]
            degacc = rest[8 + 3 * nb_]
            dsem = list(rest[9 + 3 * nb_:9 + 4 * nb_])
        else:
            (agg_out, idx_s, idx_d) = rest[:3]
            bufs = list(rest[3:3 + nb_])
            acc = rest[3 + nb_]
            gsem = list(rest[4 + nb_:4 + 2 * nb_])
            ssem = list(rest[4 + 2 * nb_:4 + 3 * nb_])

        cid = lax.axis_index("c")
        sid = lax.axis_index("s")
        row0 = sid * RPT
        is0 = cid == 0

        @pl.when(is0)
        def _():
            pltpu.sync_copy(srcL_r.at[sid], idx_s)

        @pl.when(cid == 1)
        def _():
            pltpu.sync_copy(srcR_r.at[sid], idx_s)

        pltpu.sync_copy(dst_r.at[sid], idx_d)
        pltpu.sync_copy(z, acc.at[pl.ds(row0, RPT)])
        if with_deg:
            @pl.when(is0)
            def _():
                pltpu.sync_copy(ones, ones_v)
                pltpu.sync_copy(z8, degacc.at[pl.ds(row0, RPT)])
        plsc.subcore_barrier()

        def gather(c, j):
            return pltpu.make_async_copy(y.at[idx_s.at[c]], bufs[j],
                                         gsem[j])

        def scat_start(c, j):
            pltpu.async_copy(bufs[j], acc.at[idx_d.at[c]], ssem[j],
                             add=True)

        def scat_wait(c, j):
            pltpu.make_async_copy(bufs[j], acc.at[idx_d.at[c]],
                                  ssem[j]).wait()

        def dscat_start(c, j):
            pltpu.async_copy(ones_v, degacc.at[idx_d.at[c]], dsem[j],
                             add=True)

        def dscat_wait(c, j):
            pltpu.make_async_copy(ones_v, degacc.at[idx_d.at[c]],
                                  dsem[j]).wait()

        for c in range(_PF):
            gather(c, c).start()

        def step(i, carry):
            for b in range(_NBUF):
                c = _NBUF * i + b
                gather(c, b).wait()
                scat_start(c, b)
                if with_deg:
                    @pl.when(is0)
                    def _():
                        @pl.when(c >= _NBUF)
                        def _():
                            dscat_wait(c - _NBUF, b)
                        dscat_start(c, b)
                nb = (b + _PF) % _NBUF

                @pl.when(c + _PF < C)
                def _():
                    @pl.when(c >= _PF)
                    def _():
                        scat_wait(c - _PF, nb)
                    gather(c + _PF, nb).start()
            return carry

        lax.fori_loop(0, C // _NBUF, step, 0)
        for c in range(C - _NBUF, C):
            scat_wait(c, c % _NBUF)
            if with_deg:
                @pl.when(is0)
                def _():
                    dscat_wait(c, c % _NBUF)

        plsc.subcore_barrier()
        col = cid * DH
        pltpu.sync_copy(acc.at[pl.ds(row0, RPT)],
                        agg_out.at[pl.ds(row0, RPT), pl.ds(col, DH)])
        if with_deg:
            @pl.when(is0)
            def _():
                pltpu.sync_copy(degacc.at[pl.ds(row0, RPT)],
                                deg_out.at[pl.ds(row0, RPT)])

    out = tuple(out_type) if with_deg else out_type[0]
    return pl.kernel(body, out_type=out, mesh=_mesh,
                     scratch_types=scratch, compiler_params=_sc_params)


_sc_layer1 = _sc_layer(with_deg=True)
_sc_layer2 = _sc_layer(with_deg=False)

_R = 5000


def _tc_matmul(x, w):
    def body(x_ref, w_ref, o_ref):
        o_ref[...] = jnp.dot(x_ref[...], w_ref[...],
                             preferred_element_type=jnp.float32)

    return pl.pallas_call(
        body,
        grid=(N // _R,),
        in_specs=[pl.BlockSpec((_R, D), lambda i: (i, 0)),
                  pl.BlockSpec((D, D), lambda i: (0, 0))],
        out_specs=pl.BlockSpec((_R, D), lambda i: (i, 0)),
        out_shape=jax.ShapeDtypeStruct((N, D), jnp.float32),
    )(x, w)


def _neigh(a_ref, y_ref, d_ref):
    deg = d_ref[...].sum(axis=-1) * (1.0 / DW)
    return (a_ref[...] + y_ref[...]) / (deg + 1.0)[:, None]


_agg_specs = [
    pl.BlockSpec((_R, D), lambda i: (i, 0)),
    pl.BlockSpec((_R, D), lambda i: (i, 0)),
    pl.BlockSpec((_R, DW), lambda i: (i, 0)),
    pl.BlockSpec((1, D), lambda i: (0, 0)),
]


def _tc_combine_matmul(a, y, degp, b, w):
    def body(a_ref, y_ref, d_ref, b_ref, w_ref, o_ref):
        h = jnp.maximum(_neigh(a_ref, y_ref, d_ref) + b_ref[...], 0.0)
        o_ref[...] = jnp.dot(h, w_ref[...],
                             preferred_element_type=jnp.float32)

    return pl.pallas_call(
        body,
        grid=(N // _R,),
        in_specs=_agg_specs + [pl.BlockSpec((D, D), lambda i: (0, 0))],
        out_specs=pl.BlockSpec((_R, D), lambda i: (i, 0)),
        out_shape=jax.ShapeDtypeStruct((N, D), jnp.float32),
    )(a, y, degp, b, w)


def _tc_combine(a, y, degp, b):
    def body(a_ref, y_ref, d_ref, b_ref, o_ref):
        o_ref[...] = _neigh(a_ref, y_ref, d_ref) + b_ref[...]

    return pl.pallas_call(
        body,
        grid=(N // _R,),
        in_specs=_agg_specs,
        out_specs=pl.BlockSpec((_R, D), lambda i: (i, 0)),
        out_shape=jax.ShapeDtypeStruct((N, D), jnp.float32),
    )(a, y, degp, b)


def kernel(feats, edge_index, W1, b1, W2, b2):
    pad = EP - E
    src = jnp.concatenate([edge_index[0],
                           jnp.zeros((pad,), jnp.int32)])
    dst = jnp.concatenate([edge_index[1],
                           N + (jnp.arange(pad, dtype=jnp.int32)
                                % (NP - N))]).reshape(NS, C, K)
    src_l = (2 * src).reshape(NS, C, K)
    src_r = (2 * src + 1).reshape(NS, C, K)
    z = jnp.zeros((RPT, DH), jnp.float32)
    z8 = jnp.zeros((RPT, DW), jnp.float32)
    ones = jnp.ones((K, DW), jnp.float32)
    b1r = b1.reshape(1, D)
    b2r = b2.reshape(1, D)

    y1 = _tc_matmul(feats, W1)
    a1, degp = _sc_layer1(y1.reshape(2 * N, DH), src_l, src_r, dst,
                          z, z8, ones)
    y2 = _tc_combine_matmul(a1, y1, degp, b1r, W2)
    a2 = _sc_layer2(y2.reshape(2 * N, DH), src_l, src_r, dst, z)
    return _tc_combine(a2, y2, degp, b2r)


# speedup vs baseline: 2.8289x; 2.8289x over previous
import jax
import jax.numpy as jnp
from jax import lax
from jax.experimental import pallas as pl
from jax.experimental.pallas import tpu as pltpu
from jax.experimental.pallas import tpu_sc as plsc

N = 10000
E = 320000
D = 128
DH = D // 2
NC = 2
NS = 16
EPS = E // NS
K = 125
C = EPS // K
NP = 10112
RPT = NP // NS
DW = 8
_NBUF = 4
_PF = 2

_mesh = plsc.VectorSubcoreMesh(core_axis_name="c", subcore_axis_name="s",
                               num_cores=NC, num_subcores=NS)
_sc_params = pltpu.CompilerParams(use_tc_tiling_on_sc=False)


def _sc_layer(with_deg):
    out_type = [jax.ShapeDtypeStruct((NP, D), jnp.float32)]
    scratch = [
        pltpu.VMEM((C, K), jnp.int32),
        pltpu.VMEM((C, K), jnp.int32),
    ] + [pltpu.VMEM((K, DH), jnp.float32)] * _NBUF + [
        pltpu.VMEM_SHARED((NP, DH), jnp.float32),
    ] + [pltpu.SemaphoreType.DMA] * (2 * _NBUF)
    if with_deg:
        out_type.append(jax.ShapeDtypeStruct((NP, DW), jnp.float32))
        scratch += [
            pltpu.VMEM((K, DW), jnp.float32),
            pltpu.VMEM_SHARED((NP, DW), jnp.float32),
        ] + [pltpu.SemaphoreType.DMA] * _NBUF

    def body(y, srcL_r, srcR_r, dst_r, z, *rest):
        nb_ = _NBUF
        if with_deg:
            (z8, ones, agg_out, deg_out, idx_s, idx_d) = rest[:6]
            bufs = list(rest[6:6 + nb_])
            acc = rest[6 + nb_]
            gsem = list(rest[7 + nb_:7 + 2 * nb_])
            ssem = list(rest[7 + 2 * nb_:7 + 3 * nb_])
            ones_v = rest[7 + 3 * nb_]
            degacc = rest[8 + 3 * nb_]
            dsem = list(rest[9 + 3 * nb_:9 + 4 * nb_])
        else:
            (agg_out, idx_s, idx_d) = rest[:3]
            bufs = list(rest[3:3 + nb_])
            acc = rest[3 + nb_]
            gsem = list(rest[4 + nb_:4 + 2 * nb_])
            ssem = list(rest[4 + 2 * nb_:4 + 3 * nb_])

        cid = lax.axis_index("c")
        sid = lax.axis_index("s")
        row0 = sid * RPT
        is0 = cid == 0

        @pl.when(is0)
        def _():
            pltpu.sync_copy(srcL_r.at[sid], idx_s)

        @pl.when(cid == 1)
        def _():
            pltpu.sync_copy(srcR_r.at[sid], idx_s)

        pltpu.sync_copy(dst_r.at[sid], idx_d)
        pltpu.sync_copy(z, acc.at[pl.ds(row0, RPT)])
        if with_deg:
            @pl.when(is0)
            def _():
                pltpu.sync_copy(ones, ones_v)
                pltpu.sync_copy(z8, degacc.at[pl.ds(row0, RPT)])
        plsc.subcore_barrier()

        def gather(c, j):
            return pltpu.make_async_copy(y.at[idx_s.at[c]], bufs[j],
                                         gsem[j])

        def scat_start(c, j):
            pltpu.async_copy(bufs[j], acc.at[idx_d.at[c]], ssem[j],
                             add=True)

        def scat_wait(c, j):
            pltpu.make_async_copy(bufs[j], acc.at[idx_d.at[c]],
                                  ssem[j]).wait()

        def dscat_start(c, j):
            pltpu.async_copy(ones_v, degacc.at[idx_d.at[c]], dsem[j],
                             add=True)

        def dscat_wait(c, j):
            pltpu.make_async_copy(ones_v, degacc.at[idx_d.at[c]],
                                  dsem[j]).wait()

        for c in range(_PF):
            gather(c, c).start()

        def step(i, carry):
            for b in range(_NBUF):
                c = _NBUF * i + b
                gather(c, b).wait()
                scat_start(c, b)
                if with_deg:
                    @pl.when(is0)
                    def _():
                        @pl.when(c >= _NBUF)
                        def _():
                            dscat_wait(c - _NBUF, b)
                        dscat_start(c, b)
                nb = (b + _PF) % _NBUF

                @pl.when(c + _PF < C)
                def _():
                    @pl.when(c >= _PF)
                    def _():
                        scat_wait(c - _PF, nb)
                    gather(c + _PF, nb).start()
            return carry

        lax.fori_loop(0, C // _NBUF, step, 0)
        for c in range(C - _NBUF, C):
            scat_wait(c, c % _NBUF)
            if with_deg:
                @pl.when(is0)
                def _():
                    dscat_wait(c, c % _NBUF)

        plsc.subcore_barrier()
        col = cid * DH
        pltpu.sync_copy(acc.at[pl.ds(row0, RPT)],
                        agg_out.at[pl.ds(row0, RPT), pl.ds(col, DH)])
        if with_deg:
            @pl.when(is0)
            def _():
                pltpu.sync_copy(degacc.at[pl.ds(row0, RPT)],
                                deg_out.at[pl.ds(row0, RPT)])

    out = tuple(out_type) if with_deg else out_type[0]
    return pl.kernel(body, out_type=out, mesh=_mesh,
                     scratch_types=scratch, compiler_params=_sc_params)


_sc_layer1 = _sc_layer(with_deg=True)
_sc_layer2 = _sc_layer(with_deg=False)

_R = 5000


def _tc_matmul(x, w):
    def body(x_ref, w_ref, o_ref):
        o_ref[...] = jnp.dot(x_ref[...], w_ref[...],
                             preferred_element_type=jnp.float32)

    return pl.pallas_call(
        body,
        grid=(N // _R,),
        in_specs=[pl.BlockSpec((_R, D), lambda i: (i, 0)),
                  pl.BlockSpec((D, D), lambda i: (0, 0))],
        out_specs=pl.BlockSpec((_R, D), lambda i: (i, 0)),
        out_shape=jax.ShapeDtypeStruct((N, D), jnp.float32),
    )(x, w)


def _neigh(a_ref, y_ref, d_ref):
    deg = d_ref[...].sum(axis=-1) * (1.0 / DW)
    return (a_ref[...] + y_ref[...]) / (deg + 1.0)[:, None]


_agg_specs = [
    pl.BlockSpec((_R, D), lambda i: (i, 0)),
    pl.BlockSpec((_R, D), lambda i: (i, 0)),
    pl.BlockSpec((_R, DW), lambda i: (i, 0)),
    pl.BlockSpec((1, D), lambda i: (0, 0)),
]


def _tc_combine_matmul(a, y, degp, b, w):
    def body(a_ref, y_ref, d_ref, b_ref, w_ref, o_ref):
        h = jnp.maximum(_neigh(a_ref, y_ref, d_ref) + b_ref[...], 0.0)
        o_ref[...] = jnp.dot(h, w_ref[...],
                             preferred_element_type=jnp.float32)

    return pl.pallas_call(
        body,
        grid=(N // _R,),
        in_specs=_agg_specs + [pl.BlockSpec((D, D), lambda i: (0, 0))],
        out_specs=pl.BlockSpec((_R, D), lambda i: (i, 0)),
        out_shape=jax.ShapeDtypeStruct((N, D), jnp.float32),
    )(a, y, degp, b, w)


def _tc_combine(a, y, degp, b):
    def body(a_ref, y_ref, d_ref, b_ref, o_ref):
        o_ref[...] = _neigh(a_ref, y_ref, d_ref) + b_ref[...]

    return pl.pallas_call(
        body,
        grid=(N // _R,),
        in_specs=_agg_specs,
        out_specs=pl.BlockSpec((_R, D), lambda i: (i, 0)),
        out_shape=jax.ShapeDtypeStruct((N, D), jnp.float32),
    )(a, y, degp, b)


def kernel(feats, edge_index, W1, b1, W2, b2):
    src = edge_index[0]
    dst = edge_index[1].reshape(NS, C, K)
    src_l = (2 * src).reshape(NS, C, K)
    src_r = (2 * src + 1).reshape(NS, C, K)
    z = jnp.zeros((RPT, DH), jnp.float32)
    z8 = jnp.zeros((RPT, DW), jnp.float32)
    ones = jnp.ones((K, DW), jnp.float32)
    b1r = b1.reshape(1, D)
    b2r = b2.reshape(1, D)

    y1 = _tc_matmul(feats, W1)
    a1, degp = _sc_layer1(y1.reshape(2 * N, DH), src_l, src_r, dst,
                          z, z8, ones)
    y2 = _tc_combine_matmul(a1, y1, degp, b1r, W2)
    a2 = _sc_layer2(y2.reshape(2 * N, DH), src_l, src_r, dst, z)
    return _tc_combine(a2, y2, degp, b2r)
